# baseline (device time: 29335 ns/iter reference)
import jax
import jax.numpy as jnp
from jax import lax
from jax.experimental import pallas as pl
from jax.experimental.pallas import tpu as pltpu

N_DEV = 4
M, K, N = 2048, 2048, 2048
MP = M // N_DEV
KP = K // N_DEV
NH = N // 2


def kernel(x, w_mat):

    def body(x_hbm, w_hbm, out_hbm, xf_ref, xbf_ref, gat_ref, wst_ref,
             wbf_ref, ob_ref, x_sems, w_sems, o_sems, send_sems, recv_sems):
        my = lax.axis_index("i")

        x_dmas = []
        for idx, d in enumerate((1, 2, 3, 0)):
            k = lax.rem(my + d, N_DEV)
            blk = pl.ds(k * MP, MP)
            dma = pltpu.make_async_copy(
                x_hbm.at[blk, :], xf_ref.at[blk, :], x_sems.at[idx])
            dma.start()
            x_dmas.append(dma)

        w_order = [lax.rem(my + off, N_DEV) for off in (0, 3, 1, 2)]

        def w_load(i):
            dma = pltpu.make_async_copy(
                w_hbm.at[pl.ds(w_order[i] * KP, KP), :],
                wst_ref.at[i % 2], w_sems.at[i])
            dma.start()
            return dma

        w_dmas = [w_load(0), w_load(1)]

        for d in (1, 2, 3):
            blk = pl.ds(lax.rem(my + d, N_DEV) * MP, MP)
            x_dmas[d - 1].wait()
            xbf_ref[blk, :] = xf_ref[blk, :].astype(jnp.bfloat16)
        x_dmas[3].wait()
        gat_ref[my] = xf_ref[pl.ds(my * MP, MP), :].astype(jnp.bfloat16)

        barrier_sem = pltpu.get_barrier_semaphore()
        for d in range(1, N_DEV):
            peer = lax.rem(my + d, N_DEV)
            pl.semaphore_signal(
                barrier_sem, inc=1,
                device_id=(peer,), device_id_type=pl.DeviceIdType.MESH,
            )
        pl.semaphore_wait(barrier_sem, N_DEV - 1)

        sends = []
        for i, d in enumerate((1, 2, 3)):
            peer = lax.rem(my + d, N_DEV)
            rdma = pltpu.make_async_remote_copy(
                src_ref=xbf_ref.at[pl.ds(peer * MP, MP), :],
                dst_ref=gat_ref.at[my],
                send_sem=send_sems.at[i],
                recv_sem=recv_sems.at[i],
                device_id=(peer,),
                device_id_type=pl.DeviceIdType.MESH,
            )
            rdma.start()
            sends.append(rdma)

        def w_ready(i):
            w_dmas[i].wait()
            wbf_ref[w_order[i]] = wst_ref[i % 2].astype(jnp.bfloat16)
            if i + 2 < N_DEV:
                w_dmas.append(w_load(i + 2))

        w_ready(0)
        acc = jnp.dot(
            gat_ref[my], wbf_ref[my], preferred_element_type=jnp.float32,
        )
        w_ready(1)

        def wait_recv(d):
            src = lax.rem(my - d + N_DEV, N_DEV)
            recv = pltpu.make_async_remote_copy(
                src_ref=xbf_ref.at[pl.ds(0, MP), :],
                dst_ref=gat_ref.at[src],
                send_sem=send_sems.at[d - 1],
                recv_sem=recv_sems.at[d - 1],
                device_id=(src,),
                device_id_type=pl.DeviceIdType.MESH,
            )
            recv.wait_recv()
            return src

        src = wait_recv(1)
        acc = acc + jnp.dot(
            gat_ref[src], wbf_ref[src], preferred_element_type=jnp.float32,
        )
        w_ready(2)
        src = wait_recv(3)
        acc = acc + jnp.dot(
            gat_ref[src], wbf_ref[src], preferred_element_type=jnp.float32,
        )
        w_ready(3)

        kdiag = wait_recv(2)
        o_dmas = []
        for h in range(2):
            cols = pl.ds(h * NH, NH)
            y = acc[:, h * NH:(h + 1) * NH] + jnp.dot(
                gat_ref[kdiag], wbf_ref[kdiag, :, cols],
                preferred_element_type=jnp.float32,
            )
            ob_ref[h] = y * jax.nn.sigmoid(y)
            dma = pltpu.make_async_copy(
                ob_ref.at[h], out_hbm.at[:, cols], o_sems.at[h])
            dma.start()
            o_dmas.append(dma)

        for dma in o_dmas:
            dma.wait()
        for rdma in sends:
            rdma.wait_send()

    return pl.pallas_call(
        body,
        out_shape=jax.ShapeDtypeStruct((MP, N), jnp.float32),
        in_specs=[
            pl.BlockSpec(memory_space=pl.ANY),
            pl.BlockSpec(memory_space=pl.ANY),
        ],
        out_specs=pl.BlockSpec(memory_space=pl.ANY),
        scratch_shapes=[
            pltpu.VMEM((M, KP), jnp.float32),
            pltpu.VMEM((M, KP), jnp.bfloat16),
            pltpu.VMEM((N_DEV, MP, KP), jnp.bfloat16),
            pltpu.VMEM((2, KP, N), jnp.float32),
            pltpu.VMEM((N_DEV, KP, N), jnp.bfloat16),
            pltpu.VMEM((2, MP, NH), jnp.float32),
            pltpu.SemaphoreType.DMA((N_DEV,)),
            pltpu.SemaphoreType.DMA((N_DEV,)),
            pltpu.SemaphoreType.DMA((2,)),
            pltpu.SemaphoreType.DMA((N_DEV - 1,)),
            pltpu.SemaphoreType.DMA((N_DEV - 1,)),
        ],
        compiler_params=pltpu.CompilerParams(
            collective_id=0, vmem_limit_bytes=64 * 1024 * 1024,
        ),
    )(x, w_mat)


# device time: 27291 ns/iter; 1.0749x vs baseline; 1.0749x over previous
import jax
import jax.numpy as jnp
from jax import lax
from jax.experimental import pallas as pl
from jax.experimental.pallas import tpu as pltpu

N_DEV = 4
M, K, N = 2048, 2048, 2048
MP = M // N_DEV
KP = K // N_DEV
KPH = KP // 2

_DIAG_SEMS = (1, 2)


def kernel(x, w_mat):

    def body(x_hbm, w_hbm, out_ref, xf_ref, xbf_ref, gat_ref, wst_ref,
             wbf_ref, x_sems, w_sems, send_sems, recv_sems):
        my = lax.axis_index("i")

        x_dmas = []
        for idx, d in enumerate((1, 2, 3, 0)):
            k = lax.rem(my + d, N_DEV)
            blk = pl.ds(k * MP, MP)
            dma = pltpu.make_async_copy(
                x_hbm.at[blk, :], xf_ref.at[blk, :], x_sems.at[idx])
            dma.start()
            x_dmas.append(dma)

        w_order = [lax.rem(my + off, N_DEV) for off in (0, 3, 1, 2)]

        def w_load(i):
            dma = pltpu.make_async_copy(
                w_hbm.at[pl.ds(w_order[i] * KP, KP), :],
                wst_ref.at[i % 2], w_sems.at[i])
            dma.start()
            return dma

        w_dmas = [w_load(0), w_load(1)]

        barrier_sem = pltpu.get_barrier_semaphore()
        for d in range(1, N_DEV):
            peer = lax.rem(my + d, N_DEV)
            pl.semaphore_signal(
                barrier_sem, inc=1,
                device_id=(peer,), device_id_type=pl.DeviceIdType.MESH,
            )
        pl.semaphore_wait(barrier_sem, N_DEV - 1)

        sends = []
        for i, d in enumerate((1, 2, 3)):
            peer = lax.rem(my + d, N_DEV)
            blk = pl.ds(peer * MP, MP)
            x_dmas[i].wait()
            xbf_ref[blk, :] = xf_ref[blk, :].astype(jnp.bfloat16)
            if d == 2:
                for h, sem in enumerate(_DIAG_SEMS):
                    cols = pl.ds(h * KPH, KPH)
                    rdma = pltpu.make_async_remote_copy(
                        src_ref=xbf_ref.at[blk, cols],
                        dst_ref=gat_ref.at[my, :, cols],
                        send_sem=send_sems.at[sem],
                        recv_sem=recv_sems.at[sem],
                        device_id=(peer,),
                        device_id_type=pl.DeviceIdType.MESH,
                    )
                    rdma.start()
                    sends.append(rdma)
            else:
                sem = 0 if d == 1 else 3
                rdma = pltpu.make_async_remote_copy(
                    src_ref=xbf_ref.at[blk, :],
                    dst_ref=gat_ref.at[my],
                    send_sem=send_sems.at[sem],
                    recv_sem=recv_sems.at[sem],
                    device_id=(peer,),
                    device_id_type=pl.DeviceIdType.MESH,
                )
                rdma.start()
                sends.append(rdma)

        x_dmas[3].wait()
        gat_ref[my] = xf_ref[pl.ds(my * MP, MP), :].astype(jnp.bfloat16)

        def w_ready(i):
            w_dmas[i].wait()
            wbf_ref[w_order[i]] = wst_ref[i % 2].astype(jnp.bfloat16)
            if i + 2 < N_DEV:
                w_dmas.append(w_load(i + 2))

        w_ready(0)
        acc = jnp.dot(
            gat_ref[my], wbf_ref[my], preferred_element_type=jnp.float32,
        )
        w_ready(1)

        def wait_recv(sem, src, cols=None):
            dst = gat_ref.at[src] if cols is None else gat_ref.at[src, :, cols]
            recv = pltpu.make_async_remote_copy(
                src_ref=xbf_ref.at[pl.ds(0, MP), :],
                dst_ref=dst,
                send_sem=send_sems.at[sem],
                recv_sem=recv_sems.at[sem],
                device_id=(src,),
                device_id_type=pl.DeviceIdType.MESH,
            )
            recv.wait_recv()

        src = lax.rem(my - 1 + N_DEV, N_DEV)
        wait_recv(0, src)
        acc = acc + jnp.dot(
            gat_ref[src], wbf_ref[src], preferred_element_type=jnp.float32,
        )
        w_ready(2)
        src = lax.rem(my + 1, N_DEV)
        wait_recv(3, src)
        acc = acc + jnp.dot(
            gat_ref[src], wbf_ref[src], preferred_element_type=jnp.float32,
        )
        w_ready(3)

        kd = lax.rem(my + 2, N_DEV)
        for h, sem in enumerate(_DIAG_SEMS):
            cols = pl.ds(h * KPH, KPH)
            wait_recv(sem, kd, cols)
            acc = acc + jnp.dot(
                gat_ref[kd, :, cols], wbf_ref[kd, cols, :],
                preferred_element_type=jnp.float32,
            )

        out_ref[:, :] = acc * jax.nn.sigmoid(acc)

        for rdma in sends:
            rdma.wait_send()

    return pl.pallas_call(
        body,
        out_shape=jax.ShapeDtypeStruct((MP, N), jnp.float32),
        in_specs=[
            pl.BlockSpec(memory_space=pl.ANY),
            pl.BlockSpec(memory_space=pl.ANY),
        ],
        out_specs=pl.BlockSpec(memory_space=pltpu.VMEM),
        scratch_shapes=[
            pltpu.VMEM((M, KP), jnp.float32),
            pltpu.VMEM((M, KP), jnp.bfloat16),
            pltpu.VMEM((N_DEV, MP, KP), jnp.bfloat16),
            pltpu.VMEM((2, KP, N), jnp.float32),
            pltpu.VMEM((N_DEV, KP, N), jnp.bfloat16),
            pltpu.SemaphoreType.DMA((N_DEV,)),
            pltpu.SemaphoreType.DMA((N_DEV,)),
            pltpu.SemaphoreType.DMA((N_DEV,)),
            pltpu.SemaphoreType.DMA((N_DEV,)),
        ],
        compiler_params=pltpu.CompilerParams(
            collective_id=0, vmem_limit_bytes=64 * 1024 * 1024,
        ),
    )(x, w_mat)


# device time: 27037 ns/iter; 1.0850x vs baseline; 1.0094x over previous
import jax
import jax.numpy as jnp
from jax import lax
from jax.experimental import pallas as pl
from jax.experimental.pallas import tpu as pltpu

N_DEV = 4
M, K, N = 2048, 2048, 2048
MP = M // N_DEV
KP = K // N_DEV
MPH = MP // 2

_DIAG_SEMS = (1, 2)


def kernel(x, w_mat):

    def body(x_hbm, w_hbm, out_ref, xf_ref, xbf_ref, gat_ref, wst_ref,
             wbf_ref, x_sems, w_sems, send_sems, recv_sems):
        my = lax.axis_index("i")

        x_dmas = []
        for idx, d in enumerate((1, 2, 3, 0)):
            k = lax.rem(my + d, N_DEV)
            blk = pl.ds(k * MP, MP)
            dma = pltpu.make_async_copy(
                x_hbm.at[blk, :], xf_ref.at[blk, :], x_sems.at[idx])
            dma.start()
            x_dmas.append(dma)

        w_order = [lax.rem(my + off, N_DEV) for off in (0, 3, 1, 2)]

        def w_load(i):
            dma = pltpu.make_async_copy(
                w_hbm.at[pl.ds(w_order[i] * KP, KP), :],
                wst_ref.at[i % 2], w_sems.at[i])
            dma.start()
            return dma

        w_dmas = [w_load(0), w_load(1)]

        barrier_sem = pltpu.get_barrier_semaphore()
        for d in range(1, N_DEV):
            peer = lax.rem(my + d, N_DEV)
            pl.semaphore_signal(
                barrier_sem, inc=1,
                device_id=(peer,), device_id_type=pl.DeviceIdType.MESH,
            )
        pl.semaphore_wait(barrier_sem, N_DEV - 1)

        sends = []
        for i, d in enumerate((1, 2, 3)):
            peer = lax.rem(my + d, N_DEV)
            blk = pl.ds(peer * MP, MP)
            x_dmas[i].wait()
            xbf_ref[blk, :] = xf_ref[blk, :].astype(jnp.bfloat16)
            if d == 2:
                for h, sem in enumerate(_DIAG_SEMS):
                    rows = pl.ds(peer * MP + h * MPH, MPH)
                    rdma = pltpu.make_async_remote_copy(
                        src_ref=xbf_ref.at[rows, :],
                        dst_ref=gat_ref.at[my, pl.ds(h * MPH, MPH), :],
                        send_sem=send_sems.at[sem],
                        recv_sem=recv_sems.at[sem],
                        device_id=(peer,),
                        device_id_type=pl.DeviceIdType.MESH,
                    )
                    rdma.start()
                    sends.append(rdma)
            else:
                sem = 0 if d == 1 else 3
                rdma = pltpu.make_async_remote_copy(
                    src_ref=xbf_ref.at[blk, :],
                    dst_ref=gat_ref.at[my],
                    send_sem=send_sems.at[sem],
                    recv_sem=recv_sems.at[sem],
                    device_id=(peer,),
                    device_id_type=pl.DeviceIdType.MESH,
                )
                rdma.start()
                sends.append(rdma)

        x_dmas[3].wait()
        gat_ref[my] = xf_ref[pl.ds(my * MP, MP), :].astype(jnp.bfloat16)

        def w_ready(i):
            w_dmas[i].wait()
            wbf_ref[w_order[i]] = wst_ref[i % 2].astype(jnp.bfloat16)
            if i + 2 < N_DEV:
                w_dmas.append(w_load(i + 2))

        w_ready(0)
        acc = jnp.dot(
            gat_ref[my], wbf_ref[my], preferred_element_type=jnp.float32,
        )
        w_ready(1)

        def wait_recv(sem, src, rows=None):
            dst = gat_ref.at[src] if rows is None else gat_ref.at[src, rows, :]
            recv = pltpu.make_async_remote_copy(
                src_ref=xbf_ref.at[pl.ds(0, MP), :],
                dst_ref=dst,
                send_sem=send_sems.at[sem],
                recv_sem=recv_sems.at[sem],
                device_id=(src,),
                device_id_type=pl.DeviceIdType.MESH,
            )
            recv.wait_recv()

        src = lax.rem(my - 1 + N_DEV, N_DEV)
        wait_recv(0, src)
        acc = acc + jnp.dot(
            gat_ref[src], wbf_ref[src], preferred_element_type=jnp.float32,
        )
        w_ready(2)
        src = lax.rem(my + 1, N_DEV)
        wait_recv(3, src)
        acc = acc + jnp.dot(
            gat_ref[src], wbf_ref[src], preferred_element_type=jnp.float32,
        )
        w_ready(3)

        kd = lax.rem(my + 2, N_DEV)
        for h, sem in enumerate(_DIAG_SEMS):
            rows = pl.ds(h * MPH, MPH)
            wait_recv(sem, kd, rows)
            y = acc[h * MPH:(h + 1) * MPH, :] + jnp.dot(
                gat_ref[kd, rows, :], wbf_ref[kd],
                preferred_element_type=jnp.float32,
            )
            out_ref[rows, :] = y * jax.nn.sigmoid(y)

        for rdma in sends:
            rdma.wait_send()

    return pl.pallas_call(
        body,
        out_shape=jax.ShapeDtypeStruct((MP, N), jnp.float32),
        in_specs=[
            pl.BlockSpec(memory_space=pl.ANY),
            pl.BlockSpec(memory_space=pl.ANY),
        ],
        out_specs=pl.BlockSpec(memory_space=pltpu.VMEM),
        scratch_shapes=[
            pltpu.VMEM((M, KP), jnp.float32),
            pltpu.VMEM((M, KP), jnp.bfloat16),
            pltpu.VMEM((N_DEV, MP, KP), jnp.bfloat16),
            pltpu.VMEM((2, KP, N), jnp.float32),
            pltpu.VMEM((N_DEV, KP, N), jnp.bfloat16),
            pltpu.SemaphoreType.DMA((N_DEV,)),
            pltpu.SemaphoreType.DMA((N_DEV,)),
            pltpu.SemaphoreType.DMA((N_DEV,)),
            pltpu.SemaphoreType.DMA((N_DEV,)),
        ],
        compiler_params=pltpu.CompilerParams(
            collective_id=0, vmem_limit_bytes=64 * 1024 * 1024,
        ),
    )(x, w_mat)


# device time: 25048 ns/iter; 1.1712x vs baseline; 1.0794x over previous
import jax
import jax.numpy as jnp
from jax import lax
from jax.experimental import pallas as pl
from jax.experimental.pallas import tpu as pltpu

N_DEV = 4
M, K, N = 2048, 2048, 2048
MP = M // N_DEV
KP = K // N_DEV
MPH = MP // 2


def kernel(x, w_mat):

    def body(x_hbm, w_hbm, out_ref, xf_ref, xbf_ref, gat_ref, wst_ref,
             wbf_ref, x_sems, w_sems, send_sems, recv_sems):
        my = lax.axis_index("i")

        x_dmas = []
        for idx, d in enumerate((1, 2, 3, 0)):
            k = lax.rem(my + d, N_DEV)
            blk = pl.ds(k * MP, MP)
            dma = pltpu.make_async_copy(
                x_hbm.at[blk, :], xf_ref.at[blk, :], x_sems.at[idx])
            dma.start()
            x_dmas.append(dma)

        w_order = [lax.rem(my + off, N_DEV) for off in (0, 3, 1, 2)]

        def w_load(i):
            dma = pltpu.make_async_copy(
                w_hbm.at[pl.ds(w_order[i] * KP, KP), :],
                wst_ref.at[i % 2], w_sems.at[i])
            dma.start()
            return dma

        w_dmas = [w_load(0), w_load(1)]

        barrier_sem = pltpu.get_barrier_semaphore()
        for d in range(1, N_DEV):
            peer = lax.rem(my + d, N_DEV)
            pl.semaphore_signal(
                barrier_sem, inc=1,
                device_id=(peer,), device_id_type=pl.DeviceIdType.MESH,
            )
        pl.semaphore_wait(barrier_sem, N_DEV - 1)

        sends = []
        for i, d in enumerate((1, 2, 3)):
            peer = lax.rem(my + d, N_DEV)
            blk = pl.ds(peer * MP, MP)
            x_dmas[i].wait()
            xbf_ref[blk, :] = xf_ref[blk, :].astype(jnp.bfloat16)
            for h in range(2):
                rdma = pltpu.make_async_remote_copy(
                    src_ref=xbf_ref.at[pl.ds(peer * MP + h * MPH, MPH), :],
                    dst_ref=gat_ref.at[my, pl.ds(h * MPH, MPH), :],
                    send_sem=send_sems.at[(d - 1) * 2 + h],
                    recv_sem=recv_sems.at[(d - 1) * 2 + h],
                    device_id=(peer,),
                    device_id_type=pl.DeviceIdType.MESH,
                )
                rdma.start()
                sends.append(rdma)

        x_dmas[3].wait()
        gat_ref[my] = xf_ref[pl.ds(my * MP, MP), :].astype(jnp.bfloat16)

        def w_ready(i):
            w_dmas[i].wait()
            wbf_ref[w_order[i]] = wst_ref[i % 2].astype(jnp.bfloat16)
            if i + 2 < N_DEV:
                w_dmas.append(w_load(i + 2))

        w_ready(0)
        acc = jnp.dot(
            gat_ref[my], wbf_ref[my], preferred_element_type=jnp.float32,
        )
        w_ready(1)
        w_ready(2)
        w_ready(3)

        def wait_recv(d, h, src):
            rows = pl.ds(h * MPH, MPH)
            recv = pltpu.make_async_remote_copy(
                src_ref=xbf_ref.at[pl.ds(0, MPH), :],
                dst_ref=gat_ref.at[src, rows, :],
                send_sem=send_sems.at[(d - 1) * 2 + h],
                recv_sem=recv_sems.at[(d - 1) * 2 + h],
                device_id=(src,),
                device_id_type=pl.DeviceIdType.MESH,
            )
            recv.wait_recv()

        srcs = [(d, lax.rem(my - d + N_DEV, N_DEV)) for d in (1, 3, 2)]

        half = [acc[:MPH, :], acc[MPH:, :]]
        for h in range(2):
            rows = pl.ds(h * MPH, MPH)
            acc_h = half[h]
            for d, src in srcs:
                wait_recv(d, h, src)
                acc_h = acc_h + jnp.dot(
                    gat_ref[src, rows, :], wbf_ref[src],
                    preferred_element_type=jnp.float32,
                )
            out_ref[rows, :] = acc_h * jax.nn.sigmoid(acc_h)

        for rdma in sends:
            rdma.wait_send()

    return pl.pallas_call(
        body,
        out_shape=jax.ShapeDtypeStruct((MP, N), jnp.float32),
        in_specs=[
            pl.BlockSpec(memory_space=pl.ANY),
            pl.BlockSpec(memory_space=pl.ANY),
        ],
        out_specs=pl.BlockSpec(memory_space=pltpu.VMEM),
        scratch_shapes=[
            pltpu.VMEM((M, KP), jnp.float32),
            pltpu.VMEM((M, KP), jnp.bfloat16),
            pltpu.VMEM((N_DEV, MP, KP), jnp.bfloat16),
            pltpu.VMEM((2, KP, N), jnp.float32),
            pltpu.VMEM((N_DEV, KP, N), jnp.bfloat16),
            pltpu.SemaphoreType.DMA((N_DEV,)),
            pltpu.SemaphoreType.DMA((N_DEV,)),
            pltpu.SemaphoreType.DMA((6,)),
            pltpu.SemaphoreType.DMA((6,)),
        ],
        compiler_params=pltpu.CompilerParams(
            collective_id=0, vmem_limit_bytes=64 * 1024 * 1024,
        ),
    )(x, w_mat)


# device time: 24610 ns/iter; 1.1920x vs baseline; 1.0178x over previous
import jax
import jax.numpy as jnp
from jax import lax
from jax.experimental import pallas as pl
from jax.experimental.pallas import tpu as pltpu

N_DEV = 4
M, K, N = 2048, 2048, 2048
MP = M // N_DEV
KP = K // N_DEV
NSP = 4
MPH = MP // NSP


def kernel(x, w_mat):

    def body(x_hbm, w_hbm, out_ref, xf_ref, xbf_ref, gat_ref, wst_ref,
             wbf_ref, x_sems, w_sems, send_sems, recv_sems):
        my = lax.axis_index("i")

        x_dmas = []
        for idx, d in enumerate((1, 2, 3, 0)):
            k = lax.rem(my + d, N_DEV)
            blk = pl.ds(k * MP, MP)
            dma = pltpu.make_async_copy(
                x_hbm.at[blk, :], xf_ref.at[blk, :], x_sems.at[idx])
            dma.start()
            x_dmas.append(dma)

        w_order = [lax.rem(my + off, N_DEV) for off in (0, 3, 1, 2)]

        def w_load(i):
            dma = pltpu.make_async_copy(
                w_hbm.at[pl.ds(w_order[i] * KP, KP), :],
                wst_ref.at[i % 2], w_sems.at[i])
            dma.start()
            return dma

        w_dmas = [w_load(0), w_load(1)]

        barrier_sem = pltpu.get_barrier_semaphore()
        for d in range(1, N_DEV):
            peer = lax.rem(my + d, N_DEV)
            pl.semaphore_signal(
                barrier_sem, inc=1,
                device_id=(peer,), device_id_type=pl.DeviceIdType.MESH,
            )
        pl.semaphore_wait(barrier_sem, N_DEV - 1)

        sends = []
        for i, d in enumerate((1, 2, 3)):
            peer = lax.rem(my + d, N_DEV)
            blk = pl.ds(peer * MP, MP)
            x_dmas[i].wait()
            xbf_ref[blk, :] = xf_ref[blk, :].astype(jnp.bfloat16)
            for h in range(NSP):
                rdma = pltpu.make_async_remote_copy(
                    src_ref=xbf_ref.at[pl.ds(peer * MP + h * MPH, MPH), :],
                    dst_ref=gat_ref.at[my, pl.ds(h * MPH, MPH), :],
                    send_sem=send_sems.at[(d - 1) * NSP + h],
                    recv_sem=recv_sems.at[(d - 1) * NSP + h],
                    device_id=(peer,),
                    device_id_type=pl.DeviceIdType.MESH,
                )
                rdma.start()
                sends.append(rdma)

        x_dmas[3].wait()
        gat_ref[my] = xf_ref[pl.ds(my * MP, MP), :].astype(jnp.bfloat16)

        def w_ready(i):
            w_dmas[i].wait()
            wbf_ref[w_order[i]] = wst_ref[i % 2].astype(jnp.bfloat16)
            if i + 2 < N_DEV:
                w_dmas.append(w_load(i + 2))

        w_ready(0)
        acc = jnp.dot(
            gat_ref[my], wbf_ref[my], preferred_element_type=jnp.float32,
        )
        w_ready(1)
        w_ready(2)
        w_ready(3)

        def wait_recv(d, h, src):
            rows = pl.ds(h * MPH, MPH)
            recv = pltpu.make_async_remote_copy(
                src_ref=xbf_ref.at[pl.ds(0, MPH), :],
                dst_ref=gat_ref.at[src, rows, :],
                send_sem=send_sems.at[(d - 1) * NSP + h],
                recv_sem=recv_sems.at[(d - 1) * NSP + h],
                device_id=(src,),
                device_id_type=pl.DeviceIdType.MESH,
            )
            recv.wait_recv()

        srcs = [(d, lax.rem(my - d + N_DEV, N_DEV)) for d in (1, 3, 2)]

        for h in range(NSP):
            rows = pl.ds(h * MPH, MPH)
            acc_h = acc[h * MPH:(h + 1) * MPH, :]
            for d, src in srcs:
                wait_recv(d, h, src)
                acc_h = acc_h + jnp.dot(
                    gat_ref[src, rows, :], wbf_ref[src],
                    preferred_element_type=jnp.float32,
                )
            out_ref[rows, :] = acc_h * jax.nn.sigmoid(acc_h)

        for rdma in sends:
            rdma.wait_send()

    return pl.pallas_call(
        body,
        out_shape=jax.ShapeDtypeStruct((MP, N), jnp.float32),
        in_specs=[
            pl.BlockSpec(memory_space=pl.ANY),
            pl.BlockSpec(memory_space=pl.ANY),
        ],
        out_specs=pl.BlockSpec(memory_space=pltpu.VMEM),
        scratch_shapes=[
            pltpu.VMEM((M, KP), jnp.float32),
            pltpu.VMEM((M, KP), jnp.bfloat16),
            pltpu.VMEM((N_DEV, MP, KP), jnp.bfloat16),
            pltpu.VMEM((2, KP, N), jnp.float32),
            pltpu.VMEM((N_DEV, KP, N), jnp.bfloat16),
            pltpu.SemaphoreType.DMA((N_DEV,)),
            pltpu.SemaphoreType.DMA((N_DEV,)),
            pltpu.SemaphoreType.DMA((3 * NSP,)),
            pltpu.SemaphoreType.DMA((3 * NSP,)),
        ],
        compiler_params=pltpu.CompilerParams(
            collective_id=0, vmem_limit_bytes=64 * 1024 * 1024,
        ),
    )(x, w_mat)


# device time: 24553 ns/iter; 1.1948x vs baseline; 1.0023x over previous
import jax
import jax.numpy as jnp
from jax import lax
from jax.experimental import pallas as pl
from jax.experimental.pallas import tpu as pltpu

N_DEV = 4
M, K, N = 2048, 2048, 2048
MP = M // N_DEV
KP = K // N_DEV
NSP = 4
MPH = MP // NSP


def kernel(x, w_mat):

    def body(x_hbm, w_hbm, out_ref, xf_ref, xbf_ref, gat_ref, wst_ref,
             wbf_ref, x_sems, w_sems, send_sems, recv_sems):
        my = lax.axis_index("i")

        x_dmas = []
        for idx, d in enumerate((1, 2, 3, 0)):
            k = lax.rem(my + d, N_DEV)
            blk = pl.ds(k * MP, MP)
            dma = pltpu.make_async_copy(
                x_hbm.at[blk, :], xf_ref.at[blk, :], x_sems.at[idx])
            dma.start()
            x_dmas.append(dma)

        w_order = [lax.rem(my + off, N_DEV) for off in (0, 3, 1, 2)]

        def w_load(i):
            dma = pltpu.make_async_copy(
                w_hbm.at[pl.ds(w_order[i] * KP, KP), :],
                wst_ref.at[i % 2], w_sems.at[i])
            dma.start()
            return dma

        w_dmas = [w_load(0), w_load(1)]

        barrier_sem = pltpu.get_barrier_semaphore()
        for d in range(1, N_DEV):
            peer = lax.rem(my + d, N_DEV)
            pl.semaphore_signal(
                barrier_sem, inc=1,
                device_id=(peer,), device_id_type=pl.DeviceIdType.MESH,
            )
        pl.semaphore_wait(barrier_sem, N_DEV - 1)

        sends = []
        for i, d in enumerate((1, 2, 3)):
            peer = lax.rem(my + d, N_DEV)
            blk = pl.ds(peer * MP, MP)
            x_dmas[i].wait()
            xbf_ref[blk, :] = xf_ref[blk, :].astype(jnp.bfloat16)
        for h in range(NSP):
            for d in (1, 2, 3):
                peer = lax.rem(my + d, N_DEV)
                rdma = pltpu.make_async_remote_copy(
                    src_ref=xbf_ref.at[pl.ds(peer * MP + h * MPH, MPH), :],
                    dst_ref=gat_ref.at[my, pl.ds(h * MPH, MPH), :],
                    send_sem=send_sems.at[(d - 1) * NSP + h],
                    recv_sem=recv_sems.at[(d - 1) * NSP + h],
                    device_id=(peer,),
                    device_id_type=pl.DeviceIdType.MESH,
                )
                rdma.start()
                sends.append(rdma)

        x_dmas[3].wait()
        gat_ref[my] = xf_ref[pl.ds(my * MP, MP), :].astype(jnp.bfloat16)

        def w_ready(i):
            w_dmas[i].wait()
            wbf_ref[w_order[i]] = wst_ref[i % 2].astype(jnp.bfloat16)
            if i + 2 < N_DEV:
                w_dmas.append(w_load(i + 2))

        w_ready(0)
        acc = jnp.dot(
            gat_ref[my], wbf_ref[my], preferred_element_type=jnp.float32,
        )
        w_ready(1)
        w_ready(2)
        w_ready(3)

        def wait_recv(d, h, src):
            rows = pl.ds(h * MPH, MPH)
            recv = pltpu.make_async_remote_copy(
                src_ref=xbf_ref.at[pl.ds(0, MPH), :],
                dst_ref=gat_ref.at[src, rows, :],
                send_sem=send_sems.at[(d - 1) * NSP + h],
                recv_sem=recv_sems.at[(d - 1) * NSP + h],
                device_id=(src,),
                device_id_type=pl.DeviceIdType.MESH,
            )
            recv.wait_recv()

        srcs = [(d, lax.rem(my - d + N_DEV, N_DEV)) for d in (1, 3, 2)]

        for h in range(NSP):
            rows = pl.ds(h * MPH, MPH)
            acc_h = acc[h * MPH:(h + 1) * MPH, :]
            for d, src in srcs:
                wait_recv(d, h, src)
                acc_h = acc_h + jnp.dot(
                    gat_ref[src, rows, :], wbf_ref[src],
                    preferred_element_type=jnp.float32,
                )
            out_ref[rows, :] = acc_h * jax.nn.sigmoid(acc_h)

        for rdma in sends:
            rdma.wait_send()

    return pl.pallas_call(
        body,
        out_shape=jax.ShapeDtypeStruct((MP, N), jnp.float32),
        in_specs=[
            pl.BlockSpec(memory_space=pl.ANY),
            pl.BlockSpec(memory_space=pl.ANY),
        ],
        out_specs=pl.BlockSpec(memory_space=pltpu.VMEM),
        scratch_shapes=[
            pltpu.VMEM((M, KP), jnp.float32),
            pltpu.VMEM((M, KP), jnp.bfloat16),
            pltpu.VMEM((N_DEV, MP, KP), jnp.bfloat16),
            pltpu.VMEM((2, KP, N), jnp.float32),
            pltpu.VMEM((N_DEV, KP, N), jnp.bfloat16),
            pltpu.SemaphoreType.DMA((N_DEV,)),
            pltpu.SemaphoreType.DMA((N_DEV,)),
            pltpu.SemaphoreType.DMA((3 * NSP,)),
            pltpu.SemaphoreType.DMA((3 * NSP,)),
        ],
        compiler_params=pltpu.CompilerParams(
            collective_id=0, vmem_limit_bytes=64 * 1024 * 1024,
        ),
    )(x, w_mat)
